# baseline (speedup 1.0000x reference)
import functools

import jax
import jax.numpy as jnp
from jax import lax
from jax.experimental import pallas as pl
from jax.experimental.pallas import tpu as pltpu
from jax.experimental.pallas import tpu_sc as plsc

VOCAB = 100000
BATCH = 4096
HIST = 50
N_D = 64
NC, NS = 2, 16
NW = NC * NS
BB = BATCH // NW
NBUF = 5


@functools.partial(
    pl.kernel,
    out_type=jax.ShapeDtypeStruct((HIST, BATCH, 2 * N_D), jnp.float32),
    mesh=plsc.VectorSubcoreMesh(core_axis_name="c", subcore_axis_name="s"),
    scratch_types=[
        pltpu.VMEM((HIST, BB), jnp.int32),
        pltpu.VMEM((NBUF, BB, 2 * N_D), jnp.float32),
        [pltpu.SemaphoreType.DMA] * NBUF,
        [pltpu.SemaphoreType.DMA] * NBUF,
    ],
    compiler_params=pltpu.CompilerParams(use_tc_tiling_on_sc=False),
)
def _emb_lookup(idx_hbm, table_hbm, out_hbm, idx_v, big, gsem, wsem):
    wid = lax.axis_index("s") * NC + lax.axis_index("c")
    b0 = wid * BB
    pltpu.sync_copy(idx_hbm.at[:, pl.ds(b0, BB)], idx_v)

    def gather(h, buf):
        return pltpu.make_async_copy(
            table_hbm.at[idx_v.at[h]], big.at[buf], gsem[buf])

    def write(h, buf):
        return pltpu.make_async_copy(
            big.at[buf], out_hbm.at[h].at[pl.ds(b0, BB)], wsem[buf])

    for buf in range(NBUF):
        gather(buf, buf).start()

    def body(i, _):
        for buf in range(NBUF):
            h = NBUF * i + buf
            gather(h, buf).wait()
            write(h, buf).start()

            @pl.when(h + NBUF < HIST)
            def _():
                write(h, buf).wait()
                gather(h + NBUF, buf).start()
        return ()

    lax.fori_loop(0, HIST // NBUF, body, (), unroll=False)
    for buf in range(NBUF):
        write(HIST - NBUF + buf, buf).wait()


def kernel(input, weight):
    idx = jnp.transpose(input.astype(jnp.int32))
    table = jnp.pad(weight, ((0, 0), (0, N_D)))
    out = _emb_lookup(idx, table)
    return jnp.transpose(out, (1, 0, 2))[:, :, :N_D]


# speedup vs baseline: 1.1728x; 1.1728x over previous
import functools

import jax
import jax.numpy as jnp
from jax import lax
from jax.experimental import pallas as pl
from jax.experimental.pallas import tpu as pltpu
from jax.experimental.pallas import tpu_sc as plsc

VOCAB = 100000
BATCH = 4096
HIST = 50
N_D = 64
NC, NS = 2, 16
NW = NC * NS
BB = BATCH // NW
NBUF = 5


@functools.partial(
    pl.kernel,
    out_type=jax.ShapeDtypeStruct((HIST, BATCH, 2 * N_D), jnp.float32),
    mesh=plsc.VectorSubcoreMesh(core_axis_name="c", subcore_axis_name="s"),
    scratch_types=[
        pltpu.VMEM((HIST, BB), jnp.int32),
        pltpu.VMEM((NBUF, BB, N_D), jnp.float32),
        [pltpu.SemaphoreType.DMA] * NBUF,
        [pltpu.SemaphoreType.DMA] * NBUF,
    ],
    compiler_params=pltpu.CompilerParams(use_tc_tiling_on_sc=False),
)
def _emb_lookup(idx_hbm, table_hbm, out_hbm, idx_v, big, gsem, wsem):
    wid = lax.axis_index("s") * NC + lax.axis_index("c")
    b0 = wid * BB
    pltpu.sync_copy(idx_hbm.at[:, pl.ds(b0, BB)], idx_v)

    def gather(h, buf):
        return pltpu.make_async_copy(
            table_hbm.at[idx_v.at[h]], big.at[buf], gsem[buf])

    def write(h, buf):
        return pltpu.make_async_copy(
            big.at[buf], out_hbm.at[h].at[pl.ds(b0, BB), pl.ds(0, N_D)],
            wsem[buf])

    for buf in range(NBUF):
        gather(buf, buf).start()

    def body(i, _):
        for buf in range(NBUF):
            h = NBUF * i + buf
            gather(h, buf).wait()
            write(h, buf).start()

            @pl.when(h + NBUF < HIST)
            def _():
                write(h, buf).wait()
                gather(h + NBUF, buf).start()
        return ()

    lax.fori_loop(0, HIST // NBUF, body, (), unroll=False)
    for buf in range(NBUF):
        write(HIST - NBUF + buf, buf).wait()


def kernel(input, weight):
    idx = jnp.transpose(input.astype(jnp.int32))
    out = _emb_lookup(idx, weight)
    return jnp.transpose(out, (1, 0, 2))[:, :, :N_D]
